# Initial kernel scaffold; baseline (speedup 1.0000x reference)
import jax
import jax.numpy as jnp
from jax.experimental import pallas as pl


def kernel(node_features, edge_indices, W1a, b1a, W2a, b2a, W1b, b1b, W2b, b2b):
    raise NotImplementedError("write your pallas kernel here")


# speedup vs baseline: 13.5560x; 13.5560x over previous
import functools

import jax
import jax.numpy as jnp
from jax import lax
from jax.experimental import pallas as pl
from jax.experimental.pallas import tpu as pltpu
from jax.experimental.pallas import tpu_sc as plsc

_NC = 2
_NS = 16
_NW = _NC * _NS
_K = 128
_BLK = 1024


def _cdiv(a, b):
    return (a + b - 1) // b


def _sc_mesh():
    return plsc.VectorSubcoreMesh(core_axis_name="c", subcore_axis_name="s")


def _deg_call(dstp, n_pad):
    e_pad = dstp.shape[0]
    per_w = e_pad // _NW
    nb = per_w // _K
    rows_pt = n_pad // _NS

    @functools.partial(
        pl.kernel,
        out_type=jax.ShapeDtypeStruct((_NC, n_pad), jnp.float32),
        mesh=_sc_mesh(),
        scratch_types=[
            pltpu.VMEM((_K,), jnp.int32),
            pltpu.VMEM((_K,), jnp.float32),
            pltpu.VMEM((rows_pt,), jnp.float32),
            pltpu.VMEM_SHARED((n_pad,), jnp.float32),
        ],
    )
    def k(dst_hbm, out_hbm, didx, ones, stage, acc):
        c = lax.axis_index("c")
        s = lax.axis_index("s")
        w = c * _NS + s
        one16 = jnp.ones((16,), jnp.float32)
        zero16 = jnp.zeros((16,), jnp.float32)
        for j in range(_K // 16):
            ones[pl.ds(j * 16, 16)] = one16

        def zb(t, carry):
            stage[pl.ds(t * 16, 16)] = zero16
            return carry

        lax.fori_loop(0, rows_pt // 16, zb, 0)
        pltpu.sync_copy(stage, acc.at[pl.ds(s * rows_pt, rows_pt)])
        plsc.subcore_barrier()

        def body(b, carry):
            base = w * per_w + b * _K
            pltpu.sync_copy(dst_hbm.at[pl.ds(base, _K)], didx)
            pltpu.sync_copy(ones, acc.at[didx], add=True)
            return carry

        lax.fori_loop(0, nb, body, 0)
        plsc.subcore_barrier()
        pltpu.sync_copy(acc.at[pl.ds(s * rows_pt, rows_pt)],
                        out_hbm.at[c, pl.ds(s * rows_pt, rows_pt)])

    return k(dstp)


def _prop_call(table, srcp, dstp, w_dim):
    n_pad = table.shape[0]
    e_pad = srcp.shape[0]
    per_w = e_pad // _NW
    nb = per_w // _K
    rows_pt = n_pad // _NS
    zr = 64
    chunks = w_dim // 16

    @functools.partial(
        pl.kernel,
        out_type=jax.ShapeDtypeStruct((_NC, n_pad, w_dim), jnp.float32),
        mesh=_sc_mesh(),
        scratch_types=[
            pltpu.VMEM((_K,), jnp.int32),
            pltpu.VMEM((_K,), jnp.int32),
            pltpu.VMEM((_K, w_dim), jnp.float32),
            pltpu.VMEM((zr, w_dim), jnp.float32),
            pltpu.VMEM_SHARED((n_pad, w_dim), jnp.float32),
            pltpu.SemaphoreType.DMA,
        ],
    )
    def k(tab_hbm, src_hbm, dst_hbm, out_hbm, sidx, didx, rows, zbuf, acc, sem):
        c = lax.axis_index("c")
        s = lax.axis_index("s")
        w = c * _NS + s
        zero16 = jnp.zeros((16,), jnp.float32)

        def zb(t, carry):
            zbuf[t // chunks, pl.ds((t % chunks) * 16, 16)] = zero16
            return carry

        lax.fori_loop(0, zr * chunks, zb, 0)
        for r in range(rows_pt // zr):
            pltpu.sync_copy(zbuf, acc.at[pl.ds(s * rows_pt + r * zr, zr)])
        plsc.subcore_barrier()

        def body(b, carry):
            base = w * per_w + b * _K
            pltpu.sync_copy(src_hbm.at[pl.ds(base, _K)], sidx)
            pltpu.sync_copy(dst_hbm.at[pl.ds(base, _K)], didx)
            pltpu.async_copy(tab_hbm.at[sidx], rows, sem).wait()
            pltpu.sync_copy(rows, acc.at[didx], add=True)
            return carry

        lax.fori_loop(0, nb, body, 0)
        plsc.subcore_barrier()
        pltpu.sync_copy(acc.at[pl.ds(s * rows_pt, rows_pt)],
                        out_hbm.at[c, pl.ds(s * rows_pt, rows_pt)])

    return k(table, srcp, dstp)


def _scale_call(d0, d1, x_pad):
    n_pad, d = x_pad.shape
    grid = (n_pad // _BLK,)

    def body(d0_ref, d1_ref, x_ref, dinv_ref, xs_ref):
        deg = d0_ref[...] + d1_ref[...] + 1.0
        dinv = lax.rsqrt(deg)
        dinv_ref[...] = dinv
        xs_ref[...] = x_ref[...] * dinv

    return pl.pallas_call(
        body,
        grid=grid,
        in_specs=[
            pl.BlockSpec((_BLK, 1), lambda i: (i, 0)),
            pl.BlockSpec((_BLK, 1), lambda i: (i, 0)),
            pl.BlockSpec((_BLK, d), lambda i: (i, 0)),
        ],
        out_specs=[
            pl.BlockSpec((_BLK, 1), lambda i: (i, 0)),
            pl.BlockSpec((_BLK, d), lambda i: (i, 0)),
        ],
        out_shape=[
            jax.ShapeDtypeStruct((n_pad, 1), jnp.float32),
            jax.ShapeDtypeStruct((n_pad, d), jnp.float32),
        ],
    )(d0, d1, x_pad)


def _mid_call(s0, s1, xs, dinv, w1a, b1a, w1b, b1b, w2a, w2b):
    n_pad, d = xs.shape
    h = w1a.shape[1]
    co = w2a.shape[1]
    grid = (n_pad // _BLK,)

    pad_cols = d - 2 * co

    def body(s0_ref, s1_ref, xs_ref, dv_ref, w1a_ref, b1a_ref, w1b_ref,
             b1b_ref, w2a_ref, w2b_ref, ms_ref):
        dv = dv_ref[...]
        p = (s0_ref[...] + s1_ref[...] + xs_ref[...]) * dv
        ha = jnp.maximum(
            jnp.dot(p, w1a_ref[...], preferred_element_type=jnp.float32)
            + b1a_ref[...], 0.0)
        hb = jnp.maximum(
            jnp.dot(p, w1b_ref[...], preferred_element_type=jnp.float32)
            + b1b_ref[...], 0.0)
        ma = jnp.dot(ha, w2a_ref[...], preferred_element_type=jnp.float32)
        mb = jnp.dot(hb, w2b_ref[...], preferred_element_type=jnp.float32)
        zpad = jnp.zeros((ma.shape[0], pad_cols), jnp.float32)
        ms_ref[...] = jnp.concatenate([ma, mb, zpad], axis=1) * dv

    full = lambda shape: pl.BlockSpec(shape, lambda i: tuple(0 for _ in shape))
    return pl.pallas_call(
        body,
        grid=grid,
        in_specs=[
            pl.BlockSpec((_BLK, d), lambda i: (i, 0)),
            pl.BlockSpec((_BLK, d), lambda i: (i, 0)),
            pl.BlockSpec((_BLK, d), lambda i: (i, 0)),
            pl.BlockSpec((_BLK, 1), lambda i: (i, 0)),
            full((d, h)), full((1, h)), full((d, h)), full((1, h)),
            full((h, co)), full((h, co)),
        ],
        out_specs=pl.BlockSpec((_BLK, d), lambda i: (i, 0)),
        out_shape=jax.ShapeDtypeStruct((n_pad, d), jnp.float32),
    )(s0, s1, xs, dinv, w1a, b1a, w1b, b1b, w2a, w2b)


def _final_call(t0, t1, ms, dinv, bcat):
    n_pad, c2 = ms.shape
    grid = (n_pad // _BLK,)

    def body(t0_ref, t1_ref, ms_ref, dv_ref, b_ref, q_ref):
        q_ref[...] = ((t0_ref[...] + t1_ref[...] + ms_ref[...]) * dv_ref[...]
                      + b_ref[...])

    return pl.pallas_call(
        body,
        grid=grid,
        in_specs=[
            pl.BlockSpec((_BLK, c2), lambda i: (i, 0)),
            pl.BlockSpec((_BLK, c2), lambda i: (i, 0)),
            pl.BlockSpec((_BLK, c2), lambda i: (i, 0)),
            pl.BlockSpec((_BLK, 1), lambda i: (i, 0)),
            pl.BlockSpec((1, c2), lambda i: (0, 0)),
        ],
        out_specs=pl.BlockSpec((_BLK, c2), lambda i: (i, 0)),
        out_shape=jax.ShapeDtypeStruct((n_pad, c2), jnp.float32),
    )(t0, t1, ms, dinv, bcat)


def kernel(node_features, edge_indices, W1a, b1a, W2a, b2a, W1b, b1b, W2b, b2b):
    n, d = node_features.shape
    e = edge_indices.shape[1]
    c_out = W2a.shape[1]

    n_pad = _cdiv(n, _BLK) * _BLK
    e_pad = _cdiv(e, _NW * _K) * (_NW * _K)

    padv = jnp.full((e_pad - e,), n, jnp.int32)
    srcp = jnp.concatenate([edge_indices[0], padv])
    dstp = jnp.concatenate([edge_indices[1], padv])
    x_pad = jnp.pad(node_features, ((0, n_pad - n), (0, 0)))

    deg = _deg_call(dstp, n_pad)
    dinv, xs = _scale_call(deg[0][:, None], deg[1][:, None], x_pad)
    s2 = _prop_call(xs, srcp, dstp, d)
    ms = _mid_call(s2[0], s2[1], xs, dinv, W1a, b1a.reshape(1, -1),
                   W1b, b1b.reshape(1, -1), W2a, W2b)
    t2 = _prop_call(ms, srcp, dstp, d)
    bcat = jnp.concatenate(
        [b2a, b2b, jnp.zeros((d - 2 * c_out,), jnp.float32)]).reshape(1, -1)
    q = _final_call(t2[0], t2[1], ms, dinv, bcat)
    return q[:n, :c_out], q[:n, c_out:2 * c_out]
